# Initial kernel scaffold; baseline (speedup 1.0000x reference)
import jax
import jax.numpy as jnp
from jax.experimental import pallas as pl


def kernel(x_seq, edge_index, W_embed, b_embed, W_conv0, b_conv0, W_conv1, b_conv1, bn_gamma, bn_beta, bn_mean, bn_var, W_pred, b_pred):
    raise NotImplementedError("write your pallas kernel here")


# speedup vs baseline: 15.9488x; 15.9488x over previous
import jax
import jax.numpy as jnp
from jax import lax
from jax.experimental import pallas as pl
from jax.experimental.pallas import tpu as pltpu
from jax.experimental.pallas import tpu_sc as plsc

N = 10000
NPAD = 10240
B = 8
D = 128
E = 320000
RT = 512
NT = NPAD // RT

NTILES = 16
NSC = 2
EPT = 20480
NJ = EPT // 128
EROWS = NTILES * NJ
EPAD = NTILES * EPT
RPT = NPAD // NTILES
GS = 8


def _dinv_body(deg_ref, out_ref):
    i = pl.program_id(0)
    node = lax.broadcasted_iota(jnp.int32, (RT // 128, 128), 0) * 128 \
        + lax.broadcasted_iota(jnp.int32, (RT // 128, 128), 1) + i * RT
    dm = jnp.where(node < N, lax.rsqrt(deg_ref[0] + 1.0), 0.0)
    t1 = jnp.concatenate(
        [jnp.broadcast_to(dm[k:k + 1, :], (128, 128)) for k in range(RT // 128)],
        axis=0)
    rowid = lax.broadcasted_iota(jnp.int32, (RT, 128), 0)
    lane = lax.broadcasted_iota(jnp.int32, (RT, 128), 1)
    msk = (lane == rowid % 128).astype(jnp.float32)
    ones = jnp.ones((128, 128), jnp.float32)
    out_ref[...] = jnp.dot(t1 * msk, ones, preferred_element_type=jnp.float32)


def _dinv_bcast(deg3d):
    return pl.pallas_call(
        _dinv_body,
        grid=(NT,),
        in_specs=[pl.BlockSpec((1, RT // 128, 128), lambda i: (i, 0, 0))],
        out_specs=pl.BlockSpec((RT, D), lambda i: (i, 0)),
        out_shape=jax.ShapeDtypeStruct((NPAD, D), jnp.float32),
    )(deg3d)


def _stage_a_body(x_ref, we_ref, be_ref, wc_ref, dinv_ref, hs_ref):
    h = jnp.dot(x_ref[0], we_ref[...], preferred_element_type=jnp.float32) \
        + be_ref[...]
    h = jnp.dot(h, wc_ref[...], preferred_element_type=jnp.float32)
    hs_ref[0] = dinv_ref[...] * h


def _stage_c_body(acc_ref, dinv_ref, bc0_ref, bns_ref, bnb_ref, wc1_ref, hs_ref):
    dinv = dinv_ref[...]
    t = dinv * acc_ref[0] + bc0_ref[...]
    t = t * bns_ref[...] + bnb_ref[...]
    t = jnp.maximum(t, 0.0)
    hs_ref[0] = dinv * jnp.dot(t, wc1_ref[...], preferred_element_type=jnp.float32)


def _stage_e_body(acc_ref, dinv_ref, bc1_ref, wp_ref, bp_ref, out_ref):
    t = dinv_ref[...] * acc_ref[0] + bc1_ref[...]
    r = jnp.sum(t * wp_ref[...], axis=1) + bp_ref[0, 0]
    out_ref[0] = r.reshape(RT // 128, 128)


def _dense_specs():
    deg_spec = pl.BlockSpec((RT, D), lambda b, i: (i, 0))
    vec_spec = pl.BlockSpec((1, D), lambda b, i: (0, 0))
    mat_spec = pl.BlockSpec((D, D), lambda b, i: (0, 0))
    row_spec = pl.BlockSpec((1, RT, D), lambda b, i: (b, i, 0))
    return deg_spec, vec_spec, mat_spec, row_spec


def _stage_a(x_pad, W_embed, b_embed, W_conv0, dinvb):
    deg_spec, vec_spec, mat_spec, row_spec = _dense_specs()
    return pl.pallas_call(
        _stage_a_body,
        grid=(B, NT),
        in_specs=[row_spec, mat_spec, vec_spec, mat_spec, deg_spec],
        out_specs=row_spec,
        out_shape=jax.ShapeDtypeStruct((B, NPAD, D), jnp.float32),
    )(x_pad, W_embed, b_embed.reshape(1, D), W_conv0, dinvb)


def _stage_c(acc0, dinvb, b_conv0, bn_scale, bn_shift, W_conv1):
    deg_spec, vec_spec, mat_spec, row_spec = _dense_specs()
    return pl.pallas_call(
        _stage_c_body,
        grid=(B, NT),
        in_specs=[row_spec, deg_spec, vec_spec, vec_spec, vec_spec, mat_spec],
        out_specs=row_spec,
        out_shape=jax.ShapeDtypeStruct((B, NPAD, D), jnp.float32),
    )(acc0, dinvb, b_conv0.reshape(1, D), bn_scale.reshape(1, D),
      bn_shift.reshape(1, D), W_conv1)


def _stage_e(acc1, dinvb, b_conv1, W_pred, b_pred):
    deg_spec, vec_spec, mat_spec, row_spec = _dense_specs()
    out_spec = pl.BlockSpec((1, RT // 128, 128), lambda b, i: (b * NT + i, 0, 0))
    pflat = pl.pallas_call(
        _stage_e_body,
        grid=(B, NT),
        in_specs=[row_spec, deg_spec, vec_spec, vec_spec,
                  pl.BlockSpec((1, 1), lambda b, i: (0, 0))],
        out_specs=out_spec,
        out_shape=jax.ShapeDtypeStruct((B * NT, RT // 128, 128), jnp.float32),
    )(acc1, dinvb, b_conv1.reshape(1, D), W_pred.reshape(1, D),
      b_pred.reshape(1, 1))
    return pflat.reshape(B, NPAD)[:, :N]


def _deg_body(dstm, deg_out, dst_v, ones_v, z_v, sem0, deg_sh):
    c = lax.axis_index("c")
    s = lax.axis_index("s")
    pltpu.sync_copy(dstm.at[pl.ds(s * NJ, NJ)], dst_v)
    for i in range(128 // 16):
        ones_v[pl.ds(i * 16, 16)] = jnp.full((16,), 1.0, jnp.float32)
    for i in range(RPT // 16):
        z_v[pl.ds(i * 16, 16)] = jnp.zeros((16,), jnp.float32)
    pltpu.sync_copy(z_v, deg_sh.at[pl.ds(s * RPT, RPT)])
    plsc.subcore_barrier()
    @pl.loop(0, NJ, step=8)
    def _(j):
        for i in range(8):
            pltpu.async_copy(ones_v, deg_sh.at[dst_v.at[j + i]], sem0, add=True)
        for i in range(8):
            pltpu.make_async_copy(ones_v, deg_sh.at[dst_v.at[j + i]], sem0).wait()
    plsc.subcore_barrier()

    @pl.when(c == 0)
    def _():
        pltpu.sync_copy(deg_sh.at[pl.ds(s * RPT, RPT)], z_v)
        pltpu.sync_copy(z_v, deg_out.at[pl.ds(s * RPT, RPT)])


def _degrees(dstm):
    return pl.kernel(
        _deg_body,
        out_type=jax.ShapeDtypeStruct((NPAD,), jnp.float32),
        mesh=plsc.VectorSubcoreMesh(core_axis_name="c", subcore_axis_name="s"),
        scratch_types=[
            pltpu.VMEM((NJ, 128), jnp.int32),
            pltpu.VMEM((128,), jnp.float32),
            pltpu.VMEM((RPT,), jnp.float32),
            pltpu.SemaphoreType.DMA,
            pltpu.VMEM_SHARED((NPAD,), jnp.float32),
        ],
    )(dstm)


def _mp_body(hs, srcm, dstm, out, src_v, dst_v, rows0, rows1, sem0, sem1, acc):
    c = lax.axis_index("c")
    s = lax.axis_index("s")
    for bi in range(B // NSC):
        b = c * (B // NSC) + bi
        hs_b = hs.at[b]
        out_b = out.at[b]
        for t in range(RPT // 128):
            r0 = s * RPT + t * 128
            pltpu.sync_copy(hs_b.at[pl.ds(r0, 128)], rows0)
            pltpu.sync_copy(rows0, acc.at[pl.ds(r0, 128)])
        plsc.subcore_barrier()

        @pl.loop(0, NJ // GS)
        def _(g):
            base = s * NJ + g * GS
            pltpu.sync_copy(srcm.at[pl.ds(base, GS)], src_v)
            pltpu.sync_copy(dstm.at[pl.ds(base, GS)], dst_v)
            pltpu.async_copy(hs_b.at[src_v.at[0]], rows0, sem0)
            for jj in range(GS):
                buf, sem = (rows0, sem0) if jj % 2 == 0 else (rows1, sem1)
                nbuf, nsem = (rows1, sem1) if jj % 2 == 0 else (rows0, sem0)
                if jj + 1 < GS:
                    pltpu.async_copy(hs_b.at[src_v.at[jj + 1]], nbuf, nsem)
                pltpu.make_async_copy(hs_b.at[src_v.at[jj]], buf, sem).wait()
                pltpu.sync_copy(buf, acc.at[dst_v.at[jj]], add=True)

        plsc.subcore_barrier()
        for t in range(RPT // 128):
            r0 = s * RPT + t * 128
            pltpu.sync_copy(acc.at[pl.ds(r0, 128)], rows0)
            pltpu.sync_copy(rows0, out_b.at[pl.ds(r0, 128)])
        plsc.subcore_barrier()


def _message_pass(hs, srcm, dstm):
    return pl.kernel(
        _mp_body,
        out_type=jax.ShapeDtypeStruct((B, NPAD, D), jnp.float32),
        mesh=plsc.VectorSubcoreMesh(core_axis_name="c", subcore_axis_name="s"),
        scratch_types=[
            pltpu.VMEM((GS, 128), jnp.int32),
            pltpu.VMEM((GS, 128), jnp.int32),
            pltpu.VMEM((128, D), jnp.float32),
            pltpu.VMEM((128, D), jnp.float32),
            pltpu.SemaphoreType.DMA,
            pltpu.SemaphoreType.DMA,
            pltpu.VMEM_SHARED((NPAD, D), jnp.float32),
        ],
    )(hs, srcm, dstm)


def kernel(x_seq, edge_index, W_embed, b_embed, W_conv0, b_conv0,
           W_conv1, b_conv1, bn_gamma, bn_beta, bn_mean, bn_var,
           W_pred, b_pred):
    x_pad = jnp.pad(x_seq, ((0, 0), (0, NPAD - N), (0, 0)))
    pad = jnp.full((EPAD - E,), N, jnp.int32)
    srcm = jnp.concatenate([edge_index[0], pad]).reshape(EROWS, 128)
    dstm = jnp.concatenate([edge_index[1], pad]).reshape(EROWS, 128)

    deg = _degrees(dstm)
    dinvb = _dinv_bcast(deg.reshape(NT, RT // 128, 128))

    bn_scale = bn_gamma * lax.rsqrt(bn_var + 1e-5)
    bn_shift = bn_beta - bn_mean * bn_scale

    hs0 = _stage_a(x_pad, W_embed, b_embed, W_conv0, dinvb)
    acc0 = _message_pass(hs0, srcm, dstm)
    hs1 = _stage_c(acc0, dinvb, b_conv0, bn_scale, bn_shift, W_conv1)
    acc1 = _message_pass(hs1, srcm, dstm)
    return _stage_e(acc1, dinvb, b_conv1, W_pred, b_pred)
